# Initial kernel scaffold; baseline (speedup 1.0000x reference)
import jax
import jax.numpy as jnp
from jax.experimental import pallas as pl


def kernel(bb_coord, bird_ids, lengths, anchors):
    raise NotImplementedError("write your pallas kernel here")


# speedup vs baseline: 12.3208x; 12.3208x over previous
import functools

import jax
import jax.numpy as jnp
from jax.experimental import pallas as pl
from jax.experimental.pallas import tpu as pltpu

A_LEVEL = (36864, 9216, 2304, 576, 144)
BLK = 512
PAD_LEVEL = tuple(-(-a // BLK) * BLK for a in A_LEVEL)
NBLK_LEVEL = tuple(p // BLK for p in PAD_LEVEL)
NBLK = sum(NBLK_LEVEL)
BLK_START = (0, 72, 90, 95, 97)
OFF_LEVEL = (0, 36864, 46080, 48640, 49664)
TOTAL_PAD = sum(PAD_LEVEL)
NGT_PAD = 256
MIN_IOU = 0.4
MAX_IOU = 0.5
NEG = -1e9


def _scalar_tables(blk):
    lvl = ((blk >= BLK_START[1]).astype(jnp.int32)
           + (blk >= BLK_START[2]).astype(jnp.int32)
           + (blk >= BLK_START[3]).astype(jnp.int32)
           + (blk >= BLK_START[4]).astype(jnp.int32))
    start = jnp.where(lvl == 0, BLK_START[0],
            jnp.where(lvl == 1, BLK_START[1],
            jnp.where(lvl == 2, BLK_START[2],
            jnp.where(lvl == 3, BLK_START[3], BLK_START[4]))))
    avalid = jnp.where(lvl == 0, A_LEVEL[0],
             jnp.where(lvl == 1, A_LEVEL[1],
             jnp.where(lvl == 2, A_LEVEL[2],
             jnp.where(lvl == 3, A_LEVEL[3], A_LEVEL[4]))))
    base = (blk - start) * BLK
    return lvl, base, avalid


def _body(anchors_ref, gt_sub_ref, gt_lane_ref, lab_ref, reg_ref,
          smax, sarg, slvl, *, n_groups):
    pid = pl.program_id(0)
    phase = pid // NBLK
    blk = jax.lax.rem(pid, NBLK)
    lvl, base, avalid = _scalar_tables(blk)
    lvl_f = lvl.astype(jnp.float32)

    a0 = anchors_ref[0:1, :]
    a1 = anchors_ref[1:2, :]
    a2 = anchors_ref[2:3, :]
    a3 = anchors_ref[3:4, :]
    area_a = (a2 - a0) * (a3 - a1)

    g0 = gt_sub_ref[:, 0:1]
    g1 = gt_sub_ref[:, 1:2]
    g2 = gt_sub_ref[:, 2:3]
    g3 = gt_sub_ref[:, 3:4]
    area_b = gt_sub_ref[:, 4:5]
    grp_c = gt_sub_ref[:, 5:6]
    valid_c = gt_sub_ref[:, 6:7]
    jidx_c = gt_sub_ref[:, 8:9]

    iw = jnp.clip(jnp.minimum(g2, a2) - jnp.maximum(g0, a0), 0.0, None)
    ih = jnp.clip(jnp.minimum(g3, a3) - jnp.maximum(g1, a1), 0.0, None)
    inter = iw * ih
    iou = inter / (area_a + area_b - inter + 1e-9)

    lane_iota = jax.lax.broadcasted_iota(jnp.int32, (1, BLK), 1).astype(jnp.float32)
    gidx_f = base.astype(jnp.float32) + lane_iota
    lane_ok = gidx_f < avalid.astype(jnp.float32)

    @pl.when(phase == 0)
    def _phase0():
        iou_m = jnp.where(lane_ok, iou, NEG)
        colmax = jnp.max(iou_m, axis=1, keepdims=True)
        colarg = jnp.min(jnp.where(iou_m == colmax, gidx_f, 1e9),
                         axis=1, keepdims=True)
        for L in range(5):
            @pl.when(lvl == L)
            def _upd(L=L, colmax=colmax, colarg=colarg):
                @pl.when(base == 0)
                def _init():
                    smax[:, L:L + 1] = colmax
                    sarg[:, L:L + 1] = colarg

                @pl.when(base > 0)
                def _acc():
                    cur = smax[:, L:L + 1]
                    better = colmax > cur
                    smax[:, L:L + 1] = jnp.where(better, colmax, cur)
                    sarg[:, L:L + 1] = jnp.where(better, colarg, sarg[:, L:L + 1])

    @pl.when(pid == NBLK)
    def _cross_level():
        best = smax[:, 0:1]
        lvlv = jnp.zeros_like(best)
        for L in range(1, 5):
            m = smax[:, L:L + 1] > best
            best = jnp.where(m, smax[:, L:L + 1], best)
            lvlv = jnp.where(m, float(L), lvlv)
        slvl[:, 0:1] = lvlv

    @pl.when(phase == 1)
    def _phase1():
        aw = a2 - a0
        ah = a3 - a1
        ax = a0 + 0.5 * aw
        ay = a1 + 0.5 * ah

        sargc = jnp.zeros((NGT_PAD, 1), jnp.float32)
        for L in range(5):
            sargc = jnp.where(lvl == L, sarg[:, L:L + 1], sargc)
        is_lvl = (slvl[:, 0:1] == lvl_f) & (valid_c > 0.0)
        eq_base = (sargc == gidx_f) & is_lvl

        for b in range(n_groups):
            in_b = (grp_c == float(b)) & (valid_c > 0.0)
            ov_b = jnp.where(in_b, iou, -1.0)
            max_b = jnp.max(ov_b, axis=0, keepdims=True)
            arg_b = jnp.min(jnp.where(ov_b == max_b, jidx_c, 1e9),
                            axis=0, keepdims=True)
            jstar = jnp.max(jnp.where(eq_base & in_b, jidx_c, -1.0),
                            axis=0, keepdims=True)
            forced = jstar >= 0.0
            idx = jnp.where(forced, jstar, arg_b)
            onehot = (jidx_c == idx).astype(jnp.float32)
            gath = jax.lax.dot_general(gt_lane_ref[0:5, :], onehot,
                                       (((1,), (0,)), ((), ())),
                                       precision=jax.lax.Precision.HIGHEST,
                                       preferred_element_type=jnp.float32)
            gx0 = gath[0:1]
            gy0 = gath[1:2]
            gx1 = gath[2:3]
            gy1 = gath[3:4]
            bird = gath[4:5]
            gw = gx1 - gx0
            gh = gy1 - gy0
            gx = gx0 + 0.5 * gw
            gy = gy0 + 0.5 * gh

            pos = (max_b >= MAX_IOU) | forced
            ign = (max_b >= MIN_IOU) & (max_b < MAX_IOU)
            lab = jnp.where(pos, bird, jnp.where(ign, -1.0, 0.0))
            lab_ref[b:b + 1, :] = lab.astype(jnp.int32)

            tx = jnp.where(pos, (gx - ax) / aw, 0.0)
            ty = jnp.where(pos, (gy - ay) / ah, 0.0)
            tw = jnp.where(pos, jnp.log(gw / aw), 0.0)
            th = jnp.where(pos, jnp.log(gh / ah), 0.0)
            reg_ref[b, :, :] = jnp.concatenate([tx, ty, tw, th], axis=0)


def kernel(bb_coord, bird_ids, lengths, anchors):
    n_gt = bb_coord.shape[0]
    n_groups = len(lengths)

    lens = jnp.asarray(lengths, jnp.int32)
    starts = jnp.cumsum(lens)
    jidx = jnp.arange(NGT_PAD, dtype=jnp.int32)
    grp = jnp.sum((jidx[:, None] >= starts[None, :]).astype(jnp.int32), axis=1)
    valid = (jidx < n_gt).astype(jnp.float32)

    pad_gt = NGT_PAD - n_gt
    bb = jnp.concatenate([bb_coord, jnp.zeros((pad_gt, 4), jnp.float32)], axis=0)
    bid = jnp.concatenate([bird_ids.astype(jnp.float32),
                           jnp.zeros((pad_gt,), jnp.float32)], axis=0)
    area_b = (bb[:, 2] - bb[:, 0]) * (bb[:, 3] - bb[:, 1])

    gt_sub = jnp.stack([bb[:, 0], bb[:, 1], bb[:, 2], bb[:, 3], area_b,
                        grp.astype(jnp.float32), valid, bid,
                        jidx.astype(jnp.float32),
                        jnp.zeros((NGT_PAD,), jnp.float32)], axis=1)
    gt_sub = jnp.pad(gt_sub, ((0, 0), (0, 6)))
    gt_lane = jnp.stack([bb[:, 0], bb[:, 1], bb[:, 2], bb[:, 3], bid,
                         jnp.zeros((NGT_PAD,), jnp.float32),
                         jnp.zeros((NGT_PAD,), jnp.float32),
                         jnp.zeros((NGT_PAD,), jnp.float32)], axis=0)

    pad_box = jnp.array([0.0, 0.0, 64.0, 64.0], jnp.float32)
    padded = []
    for a, p in zip(anchors, PAD_LEVEL):
        extra = p - a.shape[0]
        padded.append(jnp.concatenate(
            [a, jnp.broadcast_to(pad_box, (extra, 4))], axis=0))
    anchors_t = jnp.concatenate(padded, axis=0).T

    out_shapes = (
        jax.ShapeDtypeStruct((n_groups, TOTAL_PAD), jnp.int32),
        jax.ShapeDtypeStruct((n_groups, 4, TOTAL_PAD), jnp.float32),
    )
    labels_full, regs_full = pl.pallas_call(
        functools.partial(_body, n_groups=n_groups),
        grid=(2 * NBLK,),
        in_specs=[
            pl.BlockSpec((4, BLK), lambda i: (0, jax.lax.rem(i, NBLK))),
            pl.BlockSpec((NGT_PAD, 16), lambda i: (0, 0)),
            pl.BlockSpec((8, NGT_PAD), lambda i: (0, 0)),
        ],
        out_specs=(
            pl.BlockSpec((n_groups, BLK), lambda i: (0, jax.lax.rem(i, NBLK))),
            pl.BlockSpec((n_groups, 4, BLK), lambda i: (0, 0, jax.lax.rem(i, NBLK))),
        ),
        out_shape=out_shapes,
        scratch_shapes=[
            pltpu.VMEM((NGT_PAD, 8), jnp.float32),
            pltpu.VMEM((NGT_PAD, 8), jnp.float32),
            pltpu.VMEM((NGT_PAD, 8), jnp.float32),
        ],
    )(anchors_t, gt_sub, gt_lane)

    labels, regs = [], []
    for a_l, off in zip(A_LEVEL, OFF_LEVEL):
        labels.append(labels_full[:, off:off + a_l])
        regs.append(jnp.transpose(regs_full[:, :, off:off + a_l], (0, 2, 1)))
    return tuple(labels), tuple(regs)
